# baseline (speedup 1.0000x reference)
import jax
import jax.numpy as jnp
from jax.experimental import pallas as pl

B = 2
N = 1000
T = 16
F_IN = 64
H = 64
C = 12
R = 400


def _sigmoid(v):
    return 0.5 * jnp.tanh(0.5 * v) + 0.5


def _gate_weights(wf_ref, wb_ref, dtype):
    wf = jnp.transpose(wf_ref[...]).astype(dtype)
    wb = jnp.transpose(wb_ref[...]).astype(dtype)
    z = jnp.zeros_like(wf[:, 0:H])
    cols = []
    for i in range(3):
        cols.append(jnp.concatenate([wf[:, i * H:(i + 1) * H], z], axis=0))
        cols.append(jnp.concatenate([z, wb[:, i * H:(i + 1) * H]], axis=0))
    return jnp.concatenate(cols, axis=1)


def _gate_bias(bf_ref, bb_ref):
    bf = bf_ref[...]
    bb = bb_ref[...]
    return jnp.concatenate(
        [p for i in range(3)
         for p in (bf[:, i * H:(i + 1) * H], bb[:, i * H:(i + 1) * H])], axis=1)


def _gru_kernel(x_ref, wihf_ref, wihb_ref, whhf_ref, whhb_ref,
                bihf_ref, bihb_ref, bhhf_ref, bhhb_ref, out_ref):
    bf16 = jnp.bfloat16
    wih = _gate_weights(wihf_ref, wihb_ref, bf16)
    whh = _gate_weights(whhf_ref, whhb_ref, bf16)
    bih = _gate_bias(bihf_ref, bihb_ref)
    bhh = _gate_bias(bhhf_ref, bhhb_ref)
    xb = x_ref[...].astype(bf16)
    xcall = jnp.concatenate(
        [jnp.concatenate([xb[:, t * F_IN:(t + 1) * F_IN],
                          xb[:, (T - 1 - t) * F_IN:(T - t) * F_IN]], axis=1)
         for t in range(T)], axis=0)
    gx = jnp.dot(xcall, wih, preferred_element_type=jnp.float32) + bih
    h = jnp.zeros((R, 2 * H), jnp.float32)
    acc = jnp.zeros((R, 2 * H), jnp.float32)
    for t in range(T):
        gh = jnp.dot(h.astype(bf16), whh,
                     preferred_element_type=jnp.float32) + bhh
        gxt = gx[t * R:(t + 1) * R]
        rz = _sigmoid(gxt[:, 0:256] + gh[:, 0:256])
        r = rz[:, 0:128]
        z = rz[:, 128:256]
        n = jnp.tanh(gxt[:, 256:384] + r * gh[:, 256:384])
        h = (1.0 - z) * n + z * h
        acc = acc + h
    out_ref[...] = acc * (1.0 / T)


def _gcn_kernel(a_ref, tm_ref, w1_ref, b1_ref, w2_ref, b2_ref,
                cw_ref, cb_ref, mask_ref, out_ref):
    a = a_ref[...]
    w1 = w1_ref[...]
    w2 = w2_ref[...]
    b1 = b1_ref[...]
    b2 = b2_ref[...]
    cw = cw_ref[...]
    cb = cb_ref[0, 0]
    maskt = jnp.transpose(mask_ref[...])
    for b in range(B):
        tm = tm_ref[b]
        u1 = jnp.dot(a, tm, preferred_element_type=jnp.float32)
        h1 = jnp.maximum(jnp.dot(u1, w1, preferred_element_type=jnp.float32) + b1, 0.0)
        u2 = jnp.dot(a, h1, preferred_element_type=jnp.float32)
        h2 = jnp.maximum(jnp.dot(u2, w2, preferred_element_type=jnp.float32) + b2, 0.0)
        lg = jnp.dot(h2, cw, preferred_element_type=jnp.float32) + cb
        logits = jnp.transpose(lg)
        masked = jnp.where(maskt == 0, -1e9, logits)
        m = jnp.max(masked, axis=1, keepdims=True)
        sh = masked - m
        lse = jnp.log(jnp.sum(jnp.exp(sh), axis=1, keepdims=True))
        out_ref[b] = sh - lse


@jax.jit
def kernel(x, edges, masks, W_ih_f, W_hh_f, b_ih_f, b_hh_f,
           W_ih_b, W_hh_b, b_ih_b, b_hh_b,
           gcn1_W, gcn1_b, gcn2_W, gcn2_b, cls_W, cls_b):
    xf = x.reshape(B * N, T * F_IN)

    grid = (B * N) // R
    wspec = pl.BlockSpec((3 * H, F_IN), lambda i: (0, 0))
    hspec = pl.BlockSpec((3 * H, H), lambda i: (0, 0))
    bspec = pl.BlockSpec((1, 3 * H), lambda i: (0, 0))
    temporal = pl.pallas_call(
        _gru_kernel,
        grid=(grid,),
        in_specs=[
            pl.BlockSpec((R, T * F_IN), lambda i: (i, 0)),
            wspec, wspec, hspec, hspec, bspec, bspec, bspec, bspec,
        ],
        out_specs=pl.BlockSpec((R, 2 * H), lambda i: (i, 0)),
        out_shape=jax.ShapeDtypeStruct((B * N, 2 * H), jnp.float32),
    )(xf, W_ih_f, W_ih_b, W_hh_f, W_hh_b,
      b_ih_f.reshape(1, 3 * H), b_ih_b.reshape(1, 3 * H),
      b_hh_f.reshape(1, 3 * H), b_hh_b.reshape(1, 3 * H))

    tm = temporal.reshape(B, N, 2 * H)

    preds = pl.pallas_call(
        _gcn_kernel,
        out_shape=jax.ShapeDtypeStruct((B, C, N), jnp.float32),
    )(edges, tm, gcn1_W, gcn1_b.reshape(1, 2 * H), gcn2_W,
      gcn2_b.reshape(1, 2 * H), cls_W, cls_b.reshape(1, 1),
      masks.astype(jnp.int32))

    return preds


# speedup vs baseline: 1.4958x; 1.4958x over previous
import jax
import jax.numpy as jnp
from jax.experimental import pallas as pl

B = 2
N = 1000
T = 16
F_IN = 64
H = 64
C = 12
R = 400


def _sigmoid(v):
    return 0.5 * jnp.tanh(0.5 * v) + 0.5


def _gate_weights(wf_ref, wb_ref, dtype):
    wf = jnp.transpose(wf_ref[...]).astype(dtype)
    wb = jnp.transpose(wb_ref[...]).astype(dtype)
    z = jnp.zeros_like(wf[:, 0:H])
    cols = []
    for i in range(3):
        cols.append(jnp.concatenate([wf[:, i * H:(i + 1) * H], z], axis=0))
        cols.append(jnp.concatenate([z, wb[:, i * H:(i + 1) * H]], axis=0))
    return jnp.concatenate(cols, axis=1)


def _gate_bias(bf_ref, bb_ref):
    bf = bf_ref[...]
    bb = bb_ref[...]
    return jnp.concatenate(
        [p for i in range(3)
         for p in (bf[:, i * H:(i + 1) * H], bb[:, i * H:(i + 1) * H])], axis=1)


def _gru_kernel(xct_ref, wihf_ref, wihb_ref, whhf_ref, whhb_ref,
                bihf_ref, bihb_ref, bhhf_ref, bhhb_ref, out_ref):
    bf16 = jnp.bfloat16
    wih = _gate_weights(wihf_ref, wihb_ref, bf16)
    whh = _gate_weights(whhf_ref, whhb_ref, bf16)
    bih = _gate_bias(bihf_ref, bihb_ref)
    bhh = _gate_bias(bhhf_ref, bhhb_ref)
    xbt = xct_ref[...]
    xrev = jnp.concatenate([xbt[T - 1 - t:T - t] for t in range(T)], axis=0)
    xc = jnp.concatenate([xbt, xrev], axis=-1)
    gx = jnp.dot(xc.reshape(T * R, 2 * F_IN), wih,
                 preferred_element_type=jnp.float32) + bih
    gx = gx.reshape(T, R, 6 * H)
    h = jnp.zeros((R, 2 * H), jnp.float32)
    acc = jnp.zeros((R, 2 * H), jnp.float32)
    for t in range(T):
        gh = jnp.dot(h.astype(bf16), whh,
                     preferred_element_type=jnp.float32) + bhh
        gxt = gx[t]
        rz = _sigmoid(gxt[:, 0:256] + gh[:, 0:256])
        r = rz[:, 0:128]
        z = rz[:, 128:256]
        n = jnp.tanh(gxt[:, 256:384] + r * gh[:, 256:384])
        h = (1.0 - z) * n + z * h
        acc = acc + h
    out_ref[...] = acc * (1.0 / T)


def _gcn_kernel(a_ref, tm_ref, w1_ref, b1_ref, w2_ref, b2_ref,
                cw_ref, cb_ref, mask_ref, out_ref):
    a = a_ref[...]
    w1 = w1_ref[...]
    w2 = w2_ref[...]
    b1 = b1_ref[...]
    b2 = b2_ref[...]
    cw = cw_ref[...]
    cb = cb_ref[0, 0]
    maskt = jnp.transpose(mask_ref[...])
    for b in range(B):
        tm = tm_ref[b]
        u1 = jnp.dot(a, tm, preferred_element_type=jnp.float32)
        h1 = jnp.maximum(jnp.dot(u1, w1, preferred_element_type=jnp.float32) + b1, 0.0)
        u2 = jnp.dot(a, h1, preferred_element_type=jnp.float32)
        h2 = jnp.maximum(jnp.dot(u2, w2, preferred_element_type=jnp.float32) + b2, 0.0)
        lg = jnp.dot(h2, cw, preferred_element_type=jnp.float32) + cb
        logits = jnp.transpose(lg)
        masked = jnp.where(maskt == 0, -1e9, logits)
        m = jnp.max(masked, axis=1, keepdims=True)
        sh = masked - m
        lse = jnp.log(jnp.sum(jnp.exp(sh), axis=1, keepdims=True))
        out_ref[b] = sh - lse


@jax.jit
def kernel(x, edges, masks, W_ih_f, W_hh_f, b_ih_f, b_hh_f,
           W_ih_b, W_hh_b, b_ih_b, b_hh_b,
           gcn1_W, gcn1_b, gcn2_W, gcn2_b, cls_W, cls_b):
    xct = x.astype(jnp.bfloat16).reshape(B * N, T, F_IN).transpose(1, 0, 2)

    grid = (B * N) // R
    wspec = pl.BlockSpec((3 * H, F_IN), lambda i: (0, 0))
    hspec = pl.BlockSpec((3 * H, H), lambda i: (0, 0))
    bspec = pl.BlockSpec((1, 3 * H), lambda i: (0, 0))
    temporal = pl.pallas_call(
        _gru_kernel,
        grid=(grid,),
        in_specs=[
            pl.BlockSpec((T, R, F_IN), lambda i: (0, i, 0)),
            wspec, wspec, hspec, hspec, bspec, bspec, bspec, bspec,
        ],
        out_specs=pl.BlockSpec((R, 2 * H), lambda i: (i, 0)),
        out_shape=jax.ShapeDtypeStruct((B * N, 2 * H), jnp.float32),
    )(xct, W_ih_f, W_ih_b, W_hh_f, W_hh_b,
      b_ih_f.reshape(1, 3 * H), b_ih_b.reshape(1, 3 * H),
      b_hh_f.reshape(1, 3 * H), b_hh_b.reshape(1, 3 * H))

    tm = temporal.reshape(B, N, 2 * H)

    preds = pl.pallas_call(
        _gcn_kernel,
        out_shape=jax.ShapeDtypeStruct((B, C, N), jnp.float32),
    )(edges, tm, gcn1_W, gcn1_b.reshape(1, 2 * H), gcn2_W,
      gcn2_b.reshape(1, 2 * H), cls_W, cls_b.reshape(1, 1),
      masks.astype(jnp.int32))

    return preds
